# baseline (speedup 1.0000x reference)
import functools

import jax
import jax.numpy as jnp
from jax import lax
from jax.experimental import pallas as pl
from jax.experimental.pallas import tpu as pltpu
from jax.experimental.pallas import tpu_sc as plsc

_N = 10000
_E = 320000
_D = 128
_C = 64

_NC = 2
_NS = 16
_L = 16
_NW = _NC * _NS

_K = 128
_CPT = 80
_NQ = 5
_CPQ = _CPT // _NQ
_EPAD = _NW * _CPT * _K
_STRIPE = 632
_NPAD = _NS * _STRIPE
_DUMMY = _N

_f32 = jnp.float32


def _mesh():
    return plsc.VectorSubcoreMesh(
        core_axis_name="c", subcore_axis_name="s",
        num_cores=_NC, num_subcores=_NS)


def _seg_body(y_hbm, row_hbm, col_hbm, out_hbm, rows_v, cols_v, g0, g1,
              acc, sem0, sem1):
    cid = lax.axis_index("c")
    sid = lax.axis_index("s")
    w = cid * _NS + sid

    def zrow(i, carry):
        for t in range(_D // _L):
            g0[i, pl.ds(t * _L, _L)] = jnp.zeros((_L,), _f32)
        return carry
    lax.fori_loop(0, _K, zrow, 0)
    for z in range(_STRIPE // _K):
        pltpu.sync_copy(g0, acc.at[pl.ds(sid * _STRIPE + z * _K, _K)])
    _REM = _STRIPE % _K
    pltpu.sync_copy(g0.at[pl.ds(0, _REM)],
                    acc.at[pl.ds(sid * _STRIPE + (_STRIPE // _K) * _K, _REM)])

    plsc.subcore_barrier()

    for q in range(_NQ):
        base = w * _CPT + q * _CPQ
        pltpu.sync_copy(row_hbm.at[pl.ds(base, _CPQ)], rows_v)
        pltpu.sync_copy(col_hbm.at[pl.ds(base, _CPQ)], cols_v)

        def mrow(j, carry):
            for t in range(_K // _L):
                r = rows_v[j, pl.ds(t * _L, _L)]
                c = cols_v[j, pl.ds(t * _L, _L)]
                cols_v[j, pl.ds(t * _L, _L)] = jnp.where(
                    r == c, jnp.full((_L,), _DUMMY, jnp.int32), c)
            return carry
        lax.fori_loop(0, _CPQ, mrow, 0)

        pltpu.async_copy(y_hbm.at[rows_v.at[0]], g0, sem0)

        def step(jj, carry):
            j0 = 2 * jj
            pltpu.async_copy(y_hbm.at[rows_v.at[j0 + 1]], g1, sem1)
            pltpu.make_async_copy(y_hbm.at[rows_v.at[j0]], g0, sem0).wait()
            pltpu.sync_copy(g0, acc.at[cols_v.at[j0]], add=True)

            @pl.when(jj + 1 < _CPQ // 2)
            def _():
                pltpu.async_copy(y_hbm.at[rows_v.at[j0 + 2]], g0, sem0)
            pltpu.make_async_copy(y_hbm.at[rows_v.at[j0 + 1]], g1, sem1).wait()
            pltpu.sync_copy(g1, acc.at[cols_v.at[j0 + 1]], add=True)
            return carry
        lax.fori_loop(0, _CPQ // 2, step, 0)

    plsc.subcore_barrier()
    pltpu.sync_copy(acc.at[pl.ds(sid * _STRIPE, _STRIPE)],
                    out_hbm.at[pl.ds(cid * _NPAD + sid * _STRIPE, _STRIPE)])


def _make_seg():
    return pl.kernel(
        _seg_body,
        out_type=pltpu.HBM((_NC * _NPAD, _D), _f32),
        mesh=_mesh(),
        scratch_types=[
            pltpu.VMEM((_CPQ, _K), jnp.int32),
            pltpu.VMEM((_CPQ, _K), jnp.int32),
            pltpu.VMEM((_K, _D), _f32),
            pltpu.VMEM((_K, _D), _f32),
            pltpu.VMEM_SHARED((_NPAD, _D), _f32),
            pltpu.SemaphoreType.DMA,
            pltpu.SemaphoreType.DMA,
        ],
    )


def _cnt_body(row_hbm, col_hbm, out_hbm, rows_v, cols_v, zb, ones_v, acc):
    cid = lax.axis_index("c")
    sid = lax.axis_index("s")
    w = cid * _NS + sid

    pltpu.sync_copy(row_hbm.at[pl.ds(w * _CPT, _CPT)], rows_v)
    pltpu.sync_copy(col_hbm.at[pl.ds(w * _CPT, _CPT)], cols_v)

    def zrow(i, carry):
        zb[pl.ds(i * _L, _L)] = jnp.zeros((_L,), _f32)
        return carry
    lax.fori_loop(0, 640 // _L, zrow, 0)
    @pl.when(sid < _NS - 1)
    def _():
        pltpu.sync_copy(zb, acc.at[pl.ds(sid * 640, 640)])

    @pl.when(sid == _NS - 1)
    def _():
        pltpu.sync_copy(zb.at[pl.ds(0, 512)],
                        acc.at[pl.ds((_NS - 1) * 640, 512)])

    def orow(i, carry):
        ones_v[pl.ds(i * _L, _L)] = jnp.ones((_L,), _f32)
        return carry
    lax.fori_loop(0, _K // _L, orow, 0)

    def mrow(j, carry):
        for t in range(_K // _L):
            r = rows_v[j, pl.ds(t * _L, _L)]
            c = cols_v[j, pl.ds(t * _L, _L)]
            cols_v[j, pl.ds(t * _L, _L)] = jnp.where(
                r == c, jnp.full((_L,), _DUMMY, jnp.int32), c)
        return carry
    lax.fori_loop(0, _CPT, mrow, 0)

    plsc.subcore_barrier()

    def step(j, carry):
        pltpu.sync_copy(ones_v, acc.at[cols_v.at[j]], add=True)
        return carry
    lax.fori_loop(0, _CPT, step, 0)

    plsc.subcore_barrier()

    @pl.when(sid < _NS - 1)
    def _():
        pltpu.sync_copy(acc.at[pl.ds(sid * 640, 640)],
                        out_hbm.at[pl.ds(cid * _NPAD + sid * 640, 640)])

    @pl.when(sid == _NS - 1)
    def _():
        pltpu.sync_copy(
            acc.at[pl.ds((_NS - 1) * 640, 512)],
            out_hbm.at[pl.ds(cid * _NPAD + (_NS - 1) * 640, 512)])


def _make_cnt():
    return pl.kernel(
        _cnt_body,
        out_type=pltpu.HBM((_NC * _NPAD,), _f32),
        mesh=_mesh(),
        scratch_types=[
            pltpu.VMEM((_CPT, _K), jnp.int32),
            pltpu.VMEM((_CPT, _K), jnp.int32),
            pltpu.VMEM((640,), _f32),
            pltpu.VMEM((_K,), _f32),
            pltpu.VMEM_SHARED((_NPAD,), _f32),
        ],
    )


_DN = (((1,), (1,)), ((), ()))


def _pre_body(x_ref, w_ref, b_ref, xl_ref, y_ref):
    xb = x_ref[...]
    w = w_ref[...]
    b = b_ref[...].reshape(1, _D)
    xl = lax.dot_general(xb, w, _DN, preferred_element_type=_f32) + b
    xl_ref[...] = xl
    y_ref[...] = jnp.maximum(
        lax.dot_general(xl, w, _DN, preferred_element_type=_f32) + b, 0.0)


_BN_ROWS = 1000


def _pre_call(x, w, b):
    return pl.pallas_call(
        _pre_body,
        grid=(_N // _BN_ROWS,),
        in_specs=[
            pl.BlockSpec((_BN_ROWS, _D), lambda i: (i, 0)),
            pl.BlockSpec((_D, _D), lambda i: (0, 0)),
            pl.BlockSpec((_D,), lambda i: (0,)),
        ],
        out_specs=[
            pl.BlockSpec((_BN_ROWS, _D), lambda i: (i, 0)),
            pl.BlockSpec((_BN_ROWS, _D), lambda i: (i, 0)),
        ],
        out_shape=[jax.ShapeDtypeStruct((_N, _D), _f32)] * 2,
    )(x, w, b)


def _mid_body(s_ref, cnt_ref, y_ref, xl_ref, u_ref, g_ref, be_ref,
              w_ref, b_ref, xl2_ref, y2_ref):
    s = s_ref[:_N, :] + s_ref[_NPAD:_NPAD + _N, :] + y_ref[...]
    aggr = s / cnt_ref[...]
    u = u_ref[...]
    t = jnp.maximum(
        lax.dot_general(aggr, u[:, :_D], _DN, preferred_element_type=_f32)
        + lax.dot_general(xl_ref[...], u[:, _D:], _DN,
                          preferred_element_type=_f32), 0.0)
    m = jnp.mean(t, axis=0, keepdims=True)
    v = jnp.mean((t - m) ** 2, axis=0, keepdims=True)
    h = jnp.maximum(
        g_ref[...].reshape(1, _D) * (t - m) / jnp.sqrt(v + 1e-5)
        + be_ref[...].reshape(1, _D), 0.0)
    b = b_ref[...].reshape(1, _D)
    w = w_ref[...]
    xl2 = lax.dot_general(h, w, _DN, preferred_element_type=_f32) + b
    xl2_ref[...] = xl2
    y2_ref[...] = jnp.maximum(
        lax.dot_general(xl2, w, _DN, preferred_element_type=_f32) + b, 0.0)


def _mid_call(s, cnt, y, xl, u, g, be, w, b):
    return pl.pallas_call(
        _mid_body,
        out_shape=[jax.ShapeDtypeStruct((_N, _D), _f32)] * 2,
    )(s, cnt, y, xl, u, g, be, w, b)


def _fin_body(s_ref, cnt_ref, y_ref, xl_ref, u_ref, wl_ref, bl_ref,
              out_ref):
    s = s_ref[:_N, :] + s_ref[_NPAD:_NPAD + _N, :] + y_ref[...]
    aggr = s / cnt_ref[...]
    u = u_ref[...]
    t = jnp.maximum(
        lax.dot_general(aggr, u[:, :_D], _DN, preferred_element_type=_f32)
        + lax.dot_general(xl_ref[...], u[:, _D:], _DN,
                          preferred_element_type=_f32), 0.0)
    out_ref[...] = lax.dot_general(t, wl_ref[...], _DN,
                                   preferred_element_type=_f32) \
        + bl_ref[...].reshape(1, _C)


def _fin_call(s, cnt, y, xl, u, wl, bl):
    return pl.pallas_call(
        _fin_body,
        out_shape=jax.ShapeDtypeStruct((_N, _C), _f32),
    )(s, cnt, y, xl, u, wl, bl)


def kernel(x, edge_index, W1, b1, U1, W2, b2, U2, W3, b3, U3, gamma, beta,
           Wl, bl):
    row = edge_index[0]
    col = edge_index[1]
    pad = (jnp.arange(_EPAD - _E, dtype=jnp.int32) * 37) % _N
    row2d = jnp.concatenate([row, pad]).reshape(_NW * _CPT, _K)
    col2d = jnp.concatenate([col, pad]).reshape(_NW * _CPT, _K)

    cnt_parts = _make_cnt()(row2d, col2d)
    cnt = (cnt_parts[:_N] + cnt_parts[_NPAD:_NPAD + _N]
           + 1.0).reshape(_N, 1)

    seg = _make_seg()

    xl1, y1 = _pre_call(x, W1, b1)
    s1 = seg(y1, row2d, col2d)
    xl2, y2 = _mid_call(s1, cnt, y1, xl1, U1, gamma, beta, W2, b2)
    s2 = seg(y2, row2d, col2d)
    xl3, y3 = _mid_call(s2, cnt, y2, xl2, U2, gamma, beta, W3, b3)
    s3 = seg(y3, row2d, col2d)
    return _fin_call(s3, cnt, y3, xl3, U3, Wl, bl)


# speedup vs baseline: 1.0612x; 1.0612x over previous
import functools

import jax
import jax.numpy as jnp
from jax import lax
from jax.experimental import pallas as pl
from jax.experimental.pallas import tpu as pltpu
from jax.experimental.pallas import tpu_sc as plsc

_N = 10000
_E = 320000
_D = 128
_C = 64

_NC = 2
_NS = 16
_L = 16
_NW = _NC * _NS

_K = 128
_CPT = 80
_NQ = 2
_CPQ = _CPT // _NQ
_EPAD = _NW * _CPT * _K
_STRIPE = 632
_NPAD = _NS * _STRIPE
_DUMMY = _N

_f32 = jnp.float32


def _mesh():
    return plsc.VectorSubcoreMesh(
        core_axis_name="c", subcore_axis_name="s",
        num_cores=_NC, num_subcores=_NS)


def _seg_body(y_hbm, row_hbm, col_hbm, out_hbm, rows_v, cols_v, g0, g1,
              acc, sem0, sem1):
    cid = lax.axis_index("c")
    sid = lax.axis_index("s")
    w = cid * _NS + sid

    def zrow(i, carry):
        for t in range(_D // _L):
            g0[i, pl.ds(t * _L, _L)] = jnp.zeros((_L,), _f32)
        return carry
    lax.fori_loop(0, _K, zrow, 0)
    for z in range(_STRIPE // _K):
        pltpu.sync_copy(g0, acc.at[pl.ds(sid * _STRIPE + z * _K, _K)])
    _REM = _STRIPE % _K
    pltpu.sync_copy(g0.at[pl.ds(0, _REM)],
                    acc.at[pl.ds(sid * _STRIPE + (_STRIPE // _K) * _K, _REM)])

    plsc.subcore_barrier()

    for q in range(_NQ):
        base = w * _CPT + q * _CPQ
        pltpu.sync_copy(row_hbm.at[pl.ds(base, _CPQ)], rows_v)
        pltpu.sync_copy(col_hbm.at[pl.ds(base, _CPQ)], cols_v)

        def mrow(j, carry):
            for t in range(_K // _L):
                r = rows_v[j, pl.ds(t * _L, _L)]
                c = cols_v[j, pl.ds(t * _L, _L)]
                cols_v[j, pl.ds(t * _L, _L)] = jnp.where(
                    r == c, jnp.full((_L,), _DUMMY, jnp.int32), c)
            return carry
        lax.fori_loop(0, _CPQ, mrow, 0)

        pltpu.async_copy(y_hbm.at[rows_v.at[0]], g0, sem0)

        def step(jj, carry):
            j0 = 2 * jj
            pltpu.async_copy(y_hbm.at[rows_v.at[j0 + 1]], g1, sem1)
            pltpu.make_async_copy(y_hbm.at[rows_v.at[j0]], g0, sem0).wait()
            pltpu.sync_copy(g0, acc.at[cols_v.at[j0]], add=True)

            @pl.when(jj + 1 < _CPQ // 2)
            def _():
                pltpu.async_copy(y_hbm.at[rows_v.at[j0 + 2]], g0, sem0)
            pltpu.make_async_copy(y_hbm.at[rows_v.at[j0 + 1]], g1, sem1).wait()
            pltpu.sync_copy(g1, acc.at[cols_v.at[j0 + 1]], add=True)
            return carry
        lax.fori_loop(0, _CPQ // 2, step, 0)

    plsc.subcore_barrier()
    pltpu.sync_copy(acc.at[pl.ds(sid * _STRIPE, _STRIPE)],
                    out_hbm.at[pl.ds(cid * _NPAD + sid * _STRIPE, _STRIPE)])


def _make_seg():
    return pl.kernel(
        _seg_body,
        out_type=pltpu.HBM((_NC * _NPAD, _D), _f32),
        mesh=_mesh(),
        scratch_types=[
            pltpu.VMEM((_CPQ, _K), jnp.int32),
            pltpu.VMEM((_CPQ, _K), jnp.int32),
            pltpu.VMEM((_K, _D), _f32),
            pltpu.VMEM((_K, _D), _f32),
            pltpu.VMEM_SHARED((_NPAD, _D), _f32),
            pltpu.SemaphoreType.DMA,
            pltpu.SemaphoreType.DMA,
        ],
    )


def _cnt_body(row_hbm, col_hbm, out_hbm, rows_v, cols_v, zb, ones_v, acc):
    cid = lax.axis_index("c")
    sid = lax.axis_index("s")
    w = cid * _NS + sid

    pltpu.sync_copy(row_hbm.at[pl.ds(w * _CPT, _CPT)], rows_v)
    pltpu.sync_copy(col_hbm.at[pl.ds(w * _CPT, _CPT)], cols_v)

    def zrow(i, carry):
        zb[pl.ds(i * _L, _L)] = jnp.zeros((_L,), _f32)
        return carry
    lax.fori_loop(0, 640 // _L, zrow, 0)
    @pl.when(sid < _NS - 1)
    def _():
        pltpu.sync_copy(zb, acc.at[pl.ds(sid * 640, 640)])

    @pl.when(sid == _NS - 1)
    def _():
        pltpu.sync_copy(zb.at[pl.ds(0, 512)],
                        acc.at[pl.ds((_NS - 1) * 640, 512)])

    def orow(i, carry):
        ones_v[pl.ds(i * _L, _L)] = jnp.ones((_L,), _f32)
        return carry
    lax.fori_loop(0, _K // _L, orow, 0)

    def mrow(j, carry):
        for t in range(_K // _L):
            r = rows_v[j, pl.ds(t * _L, _L)]
            c = cols_v[j, pl.ds(t * _L, _L)]
            cols_v[j, pl.ds(t * _L, _L)] = jnp.where(
                r == c, jnp.full((_L,), _DUMMY, jnp.int32), c)
        return carry
    lax.fori_loop(0, _CPT, mrow, 0)

    plsc.subcore_barrier()

    def step(j, carry):
        pltpu.sync_copy(ones_v, acc.at[cols_v.at[j]], add=True)
        return carry
    lax.fori_loop(0, _CPT, step, 0)

    plsc.subcore_barrier()

    @pl.when(sid < _NS - 1)
    def _():
        pltpu.sync_copy(acc.at[pl.ds(sid * 640, 640)],
                        out_hbm.at[pl.ds(cid * _NPAD + sid * 640, 640)])

    @pl.when(sid == _NS - 1)
    def _():
        pltpu.sync_copy(
            acc.at[pl.ds((_NS - 1) * 640, 512)],
            out_hbm.at[pl.ds(cid * _NPAD + (_NS - 1) * 640, 512)])


def _make_cnt():
    return pl.kernel(
        _cnt_body,
        out_type=pltpu.HBM((_NC * _NPAD,), _f32),
        mesh=_mesh(),
        scratch_types=[
            pltpu.VMEM((_CPT, _K), jnp.int32),
            pltpu.VMEM((_CPT, _K), jnp.int32),
            pltpu.VMEM((640,), _f32),
            pltpu.VMEM((_K,), _f32),
            pltpu.VMEM_SHARED((_NPAD,), _f32),
        ],
    )


_DN = (((1,), (1,)), ((), ()))


def _pre_body(x_ref, w_ref, b_ref, xl_ref, y_ref):
    xb = x_ref[...]
    w = w_ref[...]
    b = b_ref[...].reshape(1, _D)
    xl = lax.dot_general(xb, w, _DN, preferred_element_type=_f32) + b
    xl_ref[...] = xl
    y_ref[...] = jnp.maximum(
        lax.dot_general(xl, w, _DN, preferred_element_type=_f32) + b, 0.0)


_BN_ROWS = 1000


def _pre_call(x, w, b):
    return pl.pallas_call(
        _pre_body,
        grid=(_N // _BN_ROWS,),
        in_specs=[
            pl.BlockSpec((_BN_ROWS, _D), lambda i: (i, 0)),
            pl.BlockSpec((_D, _D), lambda i: (0, 0)),
            pl.BlockSpec((_D,), lambda i: (0,)),
        ],
        out_specs=[
            pl.BlockSpec((_BN_ROWS, _D), lambda i: (i, 0)),
            pl.BlockSpec((_BN_ROWS, _D), lambda i: (i, 0)),
        ],
        out_shape=[jax.ShapeDtypeStruct((_N, _D), _f32)] * 2,
    )(x, w, b)


def _mid_body(s_ref, cnt_ref, y_ref, xl_ref, u_ref, g_ref, be_ref,
              w_ref, b_ref, xl2_ref, y2_ref):
    s = s_ref[:_N, :] + s_ref[_NPAD:_NPAD + _N, :] + y_ref[...]
    aggr = s / cnt_ref[...]
    u = u_ref[...]
    t = jnp.maximum(
        lax.dot_general(aggr, u[:, :_D], _DN, preferred_element_type=_f32)
        + lax.dot_general(xl_ref[...], u[:, _D:], _DN,
                          preferred_element_type=_f32), 0.0)
    m = jnp.mean(t, axis=0, keepdims=True)
    v = jnp.mean((t - m) ** 2, axis=0, keepdims=True)
    h = jnp.maximum(
        g_ref[...].reshape(1, _D) * (t - m) / jnp.sqrt(v + 1e-5)
        + be_ref[...].reshape(1, _D), 0.0)
    b = b_ref[...].reshape(1, _D)
    w = w_ref[...]
    xl2 = lax.dot_general(h, w, _DN, preferred_element_type=_f32) + b
    xl2_ref[...] = xl2
    y2_ref[...] = jnp.maximum(
        lax.dot_general(xl2, w, _DN, preferred_element_type=_f32) + b, 0.0)


def _mid_call(s, cnt, y, xl, u, g, be, w, b):
    return pl.pallas_call(
        _mid_body,
        out_shape=[jax.ShapeDtypeStruct((_N, _D), _f32)] * 2,
    )(s, cnt, y, xl, u, g, be, w, b)


def _fin_body(s_ref, cnt_ref, y_ref, xl_ref, u_ref, wl_ref, bl_ref,
              out_ref):
    s = s_ref[:_N, :] + s_ref[_NPAD:_NPAD + _N, :] + y_ref[...]
    aggr = s / cnt_ref[...]
    u = u_ref[...]
    t = jnp.maximum(
        lax.dot_general(aggr, u[:, :_D], _DN, preferred_element_type=_f32)
        + lax.dot_general(xl_ref[...], u[:, _D:], _DN,
                          preferred_element_type=_f32), 0.0)
    out_ref[...] = lax.dot_general(t, wl_ref[...], _DN,
                                   preferred_element_type=_f32) \
        + bl_ref[...].reshape(1, _C)


def _fin_call(s, cnt, y, xl, u, wl, bl):
    return pl.pallas_call(
        _fin_body,
        out_shape=jax.ShapeDtypeStruct((_N, _C), _f32),
    )(s, cnt, y, xl, u, wl, bl)


def kernel(x, edge_index, W1, b1, U1, W2, b2, U2, W3, b3, U3, gamma, beta,
           Wl, bl):
    row = edge_index[0]
    col = edge_index[1]
    pad = (jnp.arange(_EPAD - _E, dtype=jnp.int32) * 37) % _N
    row2d = jnp.concatenate([row, pad]).reshape(_NW * _CPT, _K)
    col2d = jnp.concatenate([col, pad]).reshape(_NW * _CPT, _K)

    cnt_parts = _make_cnt()(row2d, col2d)
    cnt = (cnt_parts[:_N] + cnt_parts[_NPAD:_NPAD + _N]
           + 1.0).reshape(_N, 1)

    seg = _make_seg()

    xl1, y1 = _pre_call(x, W1, b1)
    s1 = seg(y1, row2d, col2d)
    xl2, y2 = _mid_call(s1, cnt, y1, xl1, U1, gamma, beta, W2, b2)
    s2 = seg(y2, row2d, col2d)
    xl3, y3 = _mid_call(s2, cnt, y2, xl2, U2, gamma, beta, W3, b3)
    s3 = seg(y3, row2d, col2d)
    return _fin_call(s3, cnt, y3, xl3, U3, Wl, bl)


# speedup vs baseline: 1.0798x; 1.0175x over previous
import functools

import jax
import jax.numpy as jnp
from jax import lax
from jax.experimental import pallas as pl
from jax.experimental.pallas import tpu as pltpu
from jax.experimental.pallas import tpu_sc as plsc

_N = 10000
_E = 320000
_D = 128
_C = 64

_NC = 2
_NS = 16
_L = 16
_NW = _NC * _NS

_K = 128
_CPT = 80
_NQ = 2
_CPQ = _CPT // _NQ
_EPAD = _NW * _CPT * _K
_STRIPE = 632
_NPAD = _NS * _STRIPE
_DUMMY = _N

_f32 = jnp.float32


def _mesh():
    return plsc.VectorSubcoreMesh(
        core_axis_name="c", subcore_axis_name="s",
        num_cores=_NC, num_subcores=_NS)


def _seg_body(y_hbm, rc_hbm, out_hbm, rcbuf, g0, g1, acc, sem0, sem1):
    cid = lax.axis_index("c")
    sid = lax.axis_index("s")
    w = cid * _NS + sid

    def stage_and_mask(q):
        pltpu.sync_copy(rc_hbm.at[w * _NQ + q], rcbuf)

        def mrow(j, carry):
            for t in range(_K // _L):
                r = rcbuf[j, pl.ds(t * _L, _L)]
                c = rcbuf[_CPQ + j, pl.ds(t * _L, _L)]
                rcbuf[_CPQ + j, pl.ds(t * _L, _L)] = jnp.where(
                    r == c, jnp.full((_L,), _DUMMY, jnp.int32), c)
            return carry
        lax.fori_loop(0, _CPQ, mrow, 0)

    stage_and_mask(0)
    pltpu.async_copy(y_hbm.at[rcbuf.at[0]], g0, sem0)

    def zrow(i, carry):
        for t in range(_D // _L):
            g1[i, pl.ds(t * _L, _L)] = jnp.zeros((_L,), _f32)
        return carry
    lax.fori_loop(0, _K, zrow, 0)
    for z in range(_STRIPE // _K):
        pltpu.sync_copy(g1, acc.at[pl.ds(sid * _STRIPE + z * _K, _K)])
    _REM = _STRIPE % _K
    pltpu.sync_copy(g1.at[pl.ds(0, _REM)],
                    acc.at[pl.ds(sid * _STRIPE + (_STRIPE // _K) * _K, _REM)])

    plsc.subcore_barrier()

    for q in range(_NQ):
        if q > 0:
            stage_and_mask(q)
            pltpu.async_copy(y_hbm.at[rcbuf.at[0]], g0, sem0)

        def step(jj, carry):
            j0 = 2 * jj
            pltpu.async_copy(y_hbm.at[rcbuf.at[j0 + 1]], g1, sem1)
            pltpu.make_async_copy(y_hbm.at[rcbuf.at[j0]], g0, sem0).wait()
            pltpu.sync_copy(g0, acc.at[rcbuf.at[_CPQ + j0]], add=True)

            @pl.when(jj + 1 < _CPQ // 2)
            def _():
                pltpu.async_copy(y_hbm.at[rcbuf.at[j0 + 2]], g0, sem0)
            pltpu.make_async_copy(y_hbm.at[rcbuf.at[j0 + 1]], g1, sem1).wait()
            pltpu.sync_copy(g1, acc.at[rcbuf.at[_CPQ + j0 + 1]], add=True)
            return carry
        lax.fori_loop(0, _CPQ // 2, step, 0)

    plsc.subcore_barrier()
    pltpu.sync_copy(acc.at[pl.ds(sid * _STRIPE, _STRIPE)],
                    out_hbm.at[pl.ds(cid * _NPAD + sid * _STRIPE, _STRIPE)])


def _make_seg():
    return pl.kernel(
        _seg_body,
        out_type=pltpu.HBM((_NC * _NPAD, _D), _f32),
        mesh=_mesh(),
        scratch_types=[
            pltpu.VMEM((2 * _CPQ, _K), jnp.int32),
            pltpu.VMEM((_K, _D), _f32),
            pltpu.VMEM((_K, _D), _f32),
            pltpu.VMEM_SHARED((_NPAD, _D), _f32),
            pltpu.SemaphoreType.DMA,
            pltpu.SemaphoreType.DMA,
        ],
    )


def _cnt_body(row_hbm, col_hbm, out_hbm, rows_v, cols_v, zb, ones_v, acc):
    cid = lax.axis_index("c")
    sid = lax.axis_index("s")
    w = cid * _NS + sid

    pltpu.sync_copy(row_hbm.at[pl.ds(w * _CPT, _CPT)], rows_v)
    pltpu.sync_copy(col_hbm.at[pl.ds(w * _CPT, _CPT)], cols_v)

    def zrow(i, carry):
        zb[pl.ds(i * _L, _L)] = jnp.zeros((_L,), _f32)
        return carry
    lax.fori_loop(0, 640 // _L, zrow, 0)
    @pl.when(sid < _NS - 1)
    def _():
        pltpu.sync_copy(zb, acc.at[pl.ds(sid * 640, 640)])

    @pl.when(sid == _NS - 1)
    def _():
        pltpu.sync_copy(zb.at[pl.ds(0, 512)],
                        acc.at[pl.ds((_NS - 1) * 640, 512)])

    def orow(i, carry):
        ones_v[pl.ds(i * _L, _L)] = jnp.ones((_L,), _f32)
        return carry
    lax.fori_loop(0, _K // _L, orow, 0)

    def mrow(j, carry):
        for t in range(_K // _L):
            r = rows_v[j, pl.ds(t * _L, _L)]
            c = cols_v[j, pl.ds(t * _L, _L)]
            cols_v[j, pl.ds(t * _L, _L)] = jnp.where(
                r == c, jnp.full((_L,), _DUMMY, jnp.int32), c)
        return carry
    lax.fori_loop(0, _CPT, mrow, 0)

    plsc.subcore_barrier()

    def step(j, carry):
        pltpu.sync_copy(ones_v, acc.at[cols_v.at[j]], add=True)
        return carry
    lax.fori_loop(0, _CPT, step, 0)

    plsc.subcore_barrier()

    @pl.when(sid < _NS - 1)
    def _():
        pltpu.sync_copy(acc.at[pl.ds(sid * 640, 640)],
                        out_hbm.at[pl.ds(cid * _NPAD + sid * 640, 640)])

    @pl.when(sid == _NS - 1)
    def _():
        pltpu.sync_copy(
            acc.at[pl.ds((_NS - 1) * 640, 512)],
            out_hbm.at[pl.ds(cid * _NPAD + (_NS - 1) * 640, 512)])


def _make_cnt():
    return pl.kernel(
        _cnt_body,
        out_type=pltpu.HBM((_NC * _NPAD,), _f32),
        mesh=_mesh(),
        scratch_types=[
            pltpu.VMEM((_CPT, _K), jnp.int32),
            pltpu.VMEM((_CPT, _K), jnp.int32),
            pltpu.VMEM((640,), _f32),
            pltpu.VMEM((_K,), _f32),
            pltpu.VMEM_SHARED((_NPAD,), _f32),
        ],
    )


_DN = (((1,), (1,)), ((), ()))


def _pre_body(x_ref, w_ref, b_ref, xl_ref, y_ref):
    xb = x_ref[...]
    w = w_ref[...]
    b = b_ref[...].reshape(1, _D)
    xl = lax.dot_general(xb, w, _DN, preferred_element_type=_f32) + b
    xl_ref[...] = xl
    y_ref[...] = jnp.maximum(
        lax.dot_general(xl, w, _DN, preferred_element_type=_f32) + b, 0.0)


_BN_ROWS = 1000


def _pre_call(x, w, b):
    return pl.pallas_call(
        _pre_body,
        grid=(_N // _BN_ROWS,),
        in_specs=[
            pl.BlockSpec((_BN_ROWS, _D), lambda i: (i, 0)),
            pl.BlockSpec((_D, _D), lambda i: (0, 0)),
            pl.BlockSpec((_D,), lambda i: (0,)),
        ],
        out_specs=[
            pl.BlockSpec((_BN_ROWS, _D), lambda i: (i, 0)),
            pl.BlockSpec((_BN_ROWS, _D), lambda i: (i, 0)),
        ],
        out_shape=[jax.ShapeDtypeStruct((_N, _D), _f32)] * 2,
    )(x, w, b)


def _mid_body(s_ref, cnt_ref, y_ref, xl_ref, u_ref, g_ref, be_ref,
              w_ref, b_ref, xl2_ref, y2_ref):
    s = s_ref[:_N, :] + s_ref[_NPAD:_NPAD + _N, :] + y_ref[...]
    aggr = s / cnt_ref[...]
    u = u_ref[...]
    t = jnp.maximum(
        lax.dot_general(aggr, u[:, :_D], _DN, preferred_element_type=_f32)
        + lax.dot_general(xl_ref[...], u[:, _D:], _DN,
                          preferred_element_type=_f32), 0.0)
    m = jnp.mean(t, axis=0, keepdims=True)
    v = jnp.mean((t - m) ** 2, axis=0, keepdims=True)
    h = jnp.maximum(
        g_ref[...].reshape(1, _D) * (t - m) / jnp.sqrt(v + 1e-5)
        + be_ref[...].reshape(1, _D), 0.0)
    b = b_ref[...].reshape(1, _D)
    w = w_ref[...]
    xl2 = lax.dot_general(h, w, _DN, preferred_element_type=_f32) + b
    xl2_ref[...] = xl2
    y2_ref[...] = jnp.maximum(
        lax.dot_general(xl2, w, _DN, preferred_element_type=_f32) + b, 0.0)


def _mid_call(s, cnt, y, xl, u, g, be, w, b):
    return pl.pallas_call(
        _mid_body,
        out_shape=[jax.ShapeDtypeStruct((_N, _D), _f32)] * 2,
    )(s, cnt, y, xl, u, g, be, w, b)


def _fin_body(s_ref, cnt_ref, y_ref, xl_ref, u_ref, wl_ref, bl_ref,
              out_ref):
    s = s_ref[:_N, :] + s_ref[_NPAD:_NPAD + _N, :] + y_ref[...]
    aggr = s / cnt_ref[...]
    u = u_ref[...]
    t = jnp.maximum(
        lax.dot_general(aggr, u[:, :_D], _DN, preferred_element_type=_f32)
        + lax.dot_general(xl_ref[...], u[:, _D:], _DN,
                          preferred_element_type=_f32), 0.0)
    out_ref[...] = lax.dot_general(t, wl_ref[...], _DN,
                                   preferred_element_type=_f32) \
        + bl_ref[...].reshape(1, _C)


def _fin_call(s, cnt, y, xl, u, wl, bl):
    return pl.pallas_call(
        _fin_body,
        out_shape=jax.ShapeDtypeStruct((_N, _C), _f32),
    )(s, cnt, y, xl, u, wl, bl)


def kernel(x, edge_index, W1, b1, U1, W2, b2, U2, W3, b3, U3, gamma, beta,
           Wl, bl):
    row = edge_index[0]
    col = edge_index[1]
    pad = (jnp.arange(_EPAD - _E, dtype=jnp.int32) * 37) % _N
    row2d = jnp.concatenate([row, pad]).reshape(_NW * _CPT, _K)
    col2d = jnp.concatenate([col, pad]).reshape(_NW * _CPT, _K)
    rc = jnp.concatenate(
        [row2d.reshape(_NW, _NQ, _CPQ, _K), col2d.reshape(_NW, _NQ, _CPQ, _K)],
        axis=2).reshape(_NW * _NQ, 2 * _CPQ, _K)

    cnt_parts = _make_cnt()(row2d, col2d)
    cnt = (cnt_parts[:_N] + cnt_parts[_NPAD:_NPAD + _N]
           + 1.0).reshape(_N, 1)

    seg = _make_seg()

    xl1, y1 = _pre_call(x, W1, b1)
    s1 = seg(y1, rc)
    xl2, y2 = _mid_call(s1, cnt, y1, xl1, U1, gamma, beta, W2, b2)
    s2 = seg(y2, rc)
    xl3, y3 = _mid_call(s2, cnt, y2, xl2, U2, gamma, beta, W3, b3)
    s3 = seg(y3, rc)
    return _fin_call(s3, cnt, y3, xl3, U3, Wl, bl)
